# baseline (speedup 1.0000x reference)
import functools

import jax
import jax.numpy as jnp
from jax import lax
from jax.experimental import pallas as pl
from jax.experimental.pallas import tpu as pltpu
from jax.experimental.pallas import tpu_sc as plsc

N = 100000
D = 128
V = 95
CHUNK = 400
NCHUNK = N // CHUNK
NW = 32
KMAX = -(-NCHUNK // NW)
L = 16

_mesh = plsc.VectorSubcoreMesh(core_axis_name="c", subcore_axis_name="s")


@functools.partial(
    pl.kernel,
    mesh=_mesh,
    out_type=jax.ShapeDtypeStruct((N * D,), jnp.float32),
    compiler_params=pltpu.CompilerParams(needs_layout_passes=False),
    scratch_types=(
        [pltpu.VMEM((V * D,), jnp.float32),
         pltpu.VMEM((KMAX * CHUNK,), jnp.int32),
         pltpu.VMEM((CHUNK * D,), jnp.float32),
         pltpu.VMEM((CHUNK * D,), jnp.float32),
         pltpu.SemaphoreType.DMA, pltpu.SemaphoreType.DMA]
    ),
)
def _embed_lookup(idx_hbm, table_hbm, out_hbm, table_v, idx_v, rows0, rows1,
                  sem_i, sem_o):
    wid = lax.axis_index("s") * 2 + lax.axis_index("c")

    def present(k):
        return wid + NW * k < NCHUNK

    def idx_copy(k):
        base = pl.multiple_of((wid + NW * k) * CHUNK, 8)
        return pltpu.make_async_copy(
            idx_hbm.at[pl.ds(base, CHUNK)],
            idx_v.at[pl.ds(k * CHUNK, CHUNK)], sem_i)

    def out_copy(k, rv):
        base = pl.multiple_of((wid + NW * k) * (CHUNK * D), 8)
        return pltpu.make_async_copy(
            rv, out_hbm.at[pl.ds(base, CHUNK * D)], sem_o)

    lanes_d = lax.iota(jnp.int32, L) * D

    def compute_chunk(k, rv):
        idx_copy(k).wait()

        @plsc.parallel_loop(0, CHUNK // L)
        def _(g):
            v = idx_v[pl.ds(k * CHUNK + g * L, L)]
            t0 = v * D
            ab = g * (L * D) + lanes_d
            for j in range(D):
                x = plsc.load_gather(table_v, [t0 + j])
                plsc.store_scatter(rv, [ab + j], x)

    pltpu.sync_copy(table_hbm, table_v)

    def prefetch(k, c):
        @pl.when(present(k))
        def _():
            idx_copy(k).start()
        return c

    lax.fori_loop(0, KMAX, prefetch, 0)

    def pair(kk, c):
        a = 2 * kk
        b = 2 * kk + 1

        @pl.when(kk > 0)
        def _():
            out_copy(a - 2, rows0).wait()

        compute_chunk(a, rows0)
        out_copy(a, rows0).start()

        @pl.when(kk > 0)
        def _():
            out_copy(b - 2, rows1).wait()

        @pl.when(present(b))
        def _():
            compute_chunk(b, rows1)
            out_copy(b, rows1).start()

        return c

    lax.fori_loop(0, KMAX // 2, pair, 0)

    out_copy(KMAX - 2, rows0).wait()

    @pl.when(present(KMAX - 1))
    def _():
        out_copy(KMAX - 1, rows1).wait()


def kernel(atomic_num, table):
    flat = _embed_lookup(atomic_num.astype(jnp.int32), table.reshape(-1))
    return flat.reshape(N, D)


# speedup vs baseline: 4.9735x; 4.9735x over previous
import functools

import jax
import jax.numpy as jnp
from jax import lax
from jax.experimental import pallas as pl
from jax.experimental.pallas import tpu as pltpu
from jax.experimental.pallas import tpu_sc as plsc

N = 100000
D = 128
V = 95
CHUNK = 400
NBUF = 2
NCHUNK = N // CHUNK
NW = 32
KMAX = -(-NCHUNK // NW)
REP = 64

_mesh = plsc.VectorSubcoreMesh(core_axis_name="c", subcore_axis_name="s")


@functools.partial(
    pl.kernel,
    mesh=_mesh,
    out_type=jax.ShapeDtypeStruct((N, D), jnp.float32),
    scratch_types=(
        [pltpu.VMEM((CHUNK,), jnp.int32) for _ in range(KMAX)]
        + [pltpu.VMEM((CHUNK, D), jnp.float32) for _ in range(NBUF)]
        + [pltpu.SemaphoreType.DMA, pltpu.SemaphoreType.DMA,
           pltpu.SemaphoreType.DMA]
    ),
)
def _embed_lookup(idx_hbm, table_hbm, out_hbm, *refs):
    idx_v = refs[:KMAX]
    rows_v = refs[KMAX:KMAX + NBUF]
    sem_i, sem_g, sem_o = refs[KMAX + NBUF:]
    wid = lax.axis_index("s") * 2 + lax.axis_index("c")

    def cbase(k):
        return pl.multiple_of((wid + NW * k) * CHUNK, 8)

    def idx_copy(k):
        return pltpu.make_async_copy(
            idx_hbm.at[pl.ds(cbase(k), CHUNK)], idx_v[k], sem_i)

    def gather_copy(k, s):
        return pltpu.make_async_copy(
            table_hbm.at[idx_v[k]], rows_v[s], sem_g)

    def out_copy(k, s):
        return pltpu.make_async_copy(
            rows_v[s], out_hbm.at[pl.ds(cbase(k), CHUNK)], sem_o)

    def when_present(k, fn):
        if (k + 1) * NW <= NCHUNK:
            fn()
        else:
            pl.when(wid + NW * k < NCHUNK)(fn)

    def prefetch_idx(k):
        def fn():
            idx_copy(k).start()
        return fn

    def drain_and_flip(k, s):
        def fn():
            gather_copy(k, s).wait()
            out_copy(k, s).start()
        return fn

    lanes95 = lax.iota(jnp.int32, 16) * V

    def start_chunk(k, s):
        def fn():
            idx_copy(k).wait()
            iv = idx_v[k]
            c = wid + NW * k

            def off_group(g, carry):
                ph = lax.rem(c + g, 4)
                sl = pl.ds(g * 16, 16)
                iv[sl] = iv[sl] + (lanes95 + ph * (16 * V))
                return carry

            lax.fori_loop(0, CHUNK // 16, off_group, 0)
            gather_copy(k, s).start()
        return fn

    def wait_out(k, s):
        def fn():
            out_copy(k, s).wait()
        return fn

    for k in range(KMAX):
        when_present(k, prefetch_idx(k))

    for k in range(KMAX):
        s = k % NBUF
        if k >= 1:
            when_present(k - 1, drain_and_flip(k - 1, (k - 1) % NBUF))
        if k >= NBUF:
            when_present(k - NBUF, wait_out(k - NBUF, s))
        when_present(k, start_chunk(k, s))

    kl = KMAX - 1
    when_present(kl, drain_and_flip(kl, kl % NBUF))
    for k in range(max(0, KMAX - NBUF), KMAX):
        when_present(k, wait_out(k, k % NBUF))


def kernel(atomic_num, table):
    table_rep = jnp.tile(table, (REP, 1))
    return _embed_lookup(atomic_num.astype(jnp.int32), table_rep)
